# baseline (speedup 1.0000x reference)
import functools

import jax
import jax.numpy as jnp
from jax import lax
from jax.experimental import pallas as pl
from jax.experimental.pallas import tpu as pltpu
from jax.experimental.pallas import tpu_sc as plsc

SEQ = 8192
DIM = 1024

_NUM_CORES = 2
_NUM_SUBCORES = 16
_NW = _NUM_CORES * _NUM_SUBCORES

_TROWS = 4096
_TROWS_PER_W = _TROWS // _NW
_TCHUNK = 32
_TNBUF = 2
_TNCHUNK = _TROWS_PER_W // _TCHUNK

_SROWS_PER_C = (SEQ - _TROWS) // _NUM_CORES
_SCHUNK = 256
_SNBUF = 3
_SNCHUNK = _SROWS_PER_C // _SCHUNK

_mesh = plsc.VectorSubcoreMesh(core_axis_name="c", subcore_axis_name="s")


def _ring_copy(src_hbm, dst_hbm, base, chunk, nchunk, bufs, isems, osems):
    nbuf = len(bufs)

    def in_copy(i):
        return pltpu.async_copy(
            src_hbm.at[pl.ds(base + i * chunk, chunk)],
            bufs[i % nbuf],
            isems[i % nbuf],
        )

    def out_copy(i):
        return pltpu.async_copy(
            bufs[i % nbuf],
            dst_hbm.at[pl.ds(base + i * chunk, chunk)],
            osems[i % nbuf],
        )

    ins = [None] * nchunk
    outs = [None] * nchunk
    for i in range(min(nbuf, nchunk)):
        ins[i] = in_copy(i)
    for i in range(nchunk):
        ins[i].wait()
        outs[i] = out_copy(i)
        nxt = i + nbuf
        if nxt < nchunk:
            outs[i].wait()
            ins[nxt] = in_copy(nxt)
    for i in range(max(0, nchunk - nbuf), nchunk):
        outs[i].wait()


@functools.partial(
    pl.kernel,
    mesh=_mesh,
    out_type=jax.ShapeDtypeStruct((SEQ, DIM), jnp.float32),
    scratch_types=(
        [pltpu.VMEM((_TCHUNK, DIM), jnp.float32) for _ in range(_TNBUF)]
        + [pltpu.VMEM_SHARED((_SCHUNK, DIM), jnp.float32) for _ in range(_SNBUF)]
        + [pltpu.SemaphoreType.DMA for _ in range(2 * _TNBUF + 2 * _SNBUF)]
    ),
)
def _sc_copy(embed_hbm, out_hbm, *scratch):
    tbufs = scratch[:_TNBUF]
    sbufs = scratch[_TNBUF:_TNBUF + _SNBUF]
    sems = scratch[_TNBUF + _SNBUF:]
    tisems = sems[:_TNBUF]
    tosems = sems[_TNBUF:2 * _TNBUF]
    sisems = sems[2 * _TNBUF:2 * _TNBUF + _SNBUF]
    sosems = sems[2 * _TNBUF + _SNBUF:]

    sid = lax.axis_index("s")
    cid = lax.axis_index("c")
    wid = sid * _NUM_CORES + cid

    @pl.when(sid == 0)
    def _spmem_path():
        _ring_copy(embed_hbm, out_hbm, _TROWS + cid * _SROWS_PER_C,
                   _SCHUNK, _SNCHUNK, sbufs, sisems, sosems)

    _ring_copy(embed_hbm, out_hbm, wid * _TROWS_PER_W,
               _TCHUNK, _TNCHUNK, tbufs, tisems, tosems)


def kernel(input, embed):
    return _sc_copy(embed).reshape(SEQ, 1, DIM)


# speedup vs baseline: 1.1145x; 1.1145x over previous
import functools

import jax
import jax.numpy as jnp
from jax import lax
from jax.experimental import pallas as pl
from jax.experimental.pallas import tpu as pltpu
from jax.experimental.pallas import tpu_sc as plsc

SEQ = 8192
DIM = 1024

_NUM_CORES = 2
_NUM_SUBCORES = 16
_NW = _NUM_CORES * _NUM_SUBCORES
_ROWS_PER_W = SEQ // _NW
_CHUNK = 16
_NBUF = 4
_NCHUNK = _ROWS_PER_W // _CHUNK

_mesh = plsc.VectorSubcoreMesh(core_axis_name="c", subcore_axis_name="s")


@functools.partial(
    pl.kernel,
    mesh=_mesh,
    out_type=jax.ShapeDtypeStruct((SEQ, DIM), jnp.float32),
    scratch_types=(
        [pltpu.VMEM((_CHUNK, DIM), jnp.float32) for _ in range(_NBUF)]
        + [pltpu.SemaphoreType.DMA for _ in range(2 * _NBUF)]
    ),
)
def _sc_copy(embed_hbm, out_hbm, *scratch):
    bufs = scratch[:_NBUF]
    isems = scratch[_NBUF:2 * _NBUF]
    osems = scratch[2 * _NBUF:]
    wid = lax.axis_index("s") * _NUM_CORES + lax.axis_index("c")
    base = wid * _ROWS_PER_W

    def in_copy(i):
        return pltpu.async_copy(
            embed_hbm.at[pl.ds(base + i * _CHUNK, _CHUNK)],
            bufs[i % _NBUF],
            isems[i % _NBUF],
        )

    def out_copy(i):
        return pltpu.async_copy(
            bufs[i % _NBUF],
            out_hbm.at[pl.ds(base + i * _CHUNK, _CHUNK)],
            osems[i % _NBUF],
        )

    ins = [None] * _NCHUNK
    outs = [None] * _NCHUNK
    for i in range(min(_NBUF, _NCHUNK)):
        ins[i] = in_copy(i)
    for i in range(_NCHUNK):
        ins[i].wait()
        outs[i] = out_copy(i)
        nxt = i + _NBUF
        if nxt < _NCHUNK:
            outs[i].wait()
            ins[nxt] = in_copy(nxt)
    for i in range(max(0, _NCHUNK - _NBUF), _NCHUNK):
        outs[i].wait()


def kernel(input, embed):
    return _sc_copy(embed).reshape(SEQ, 1, DIM)


# speedup vs baseline: 1.6879x; 1.5146x over previous
import functools

import jax
import jax.numpy as jnp
from jax import lax
from jax.experimental import pallas as pl
from jax.experimental.pallas import tpu as pltpu
from jax.experimental.pallas import tpu_sc as plsc

SEQ = 8192
DIM = 1024

_NUM_CORES = 2
_NUM_SUBCORES = 16
_NW = _NUM_CORES * _NUM_SUBCORES
_ROWS_PER_W = SEQ // _NW
_CHUNK = 16
_NBUF = 4
_NCHUNK = _ROWS_PER_W // _CHUNK

_mesh = plsc.VectorSubcoreMesh(core_axis_name="c", subcore_axis_name="s")


@functools.partial(
    pl.kernel,
    mesh=_mesh,
    out_type=jax.ShapeDtypeStruct((SEQ, 1, DIM), jnp.float32),
    scratch_types=(
        [pltpu.VMEM((_CHUNK, 1, DIM), jnp.float32) for _ in range(_NBUF)]
        + [pltpu.SemaphoreType.DMA for _ in range(2 * _NBUF)]
    ),
)
def _sc_copy(embed_hbm, out_hbm, *scratch):
    bufs = scratch[:_NBUF]
    isems = scratch[_NBUF:2 * _NBUF]
    osems = scratch[2 * _NBUF:]
    wid = lax.axis_index("s") * _NUM_CORES + lax.axis_index("c")
    base = wid * _ROWS_PER_W

    def in_copy(i):
        return pltpu.async_copy(
            embed_hbm.at[pl.ds(base + i * _CHUNK, _CHUNK)],
            bufs[i % _NBUF].at[:, 0, :],
            isems[i % _NBUF],
        )

    def out_copy(i):
        return pltpu.async_copy(
            bufs[i % _NBUF],
            out_hbm.at[pl.ds(base + i * _CHUNK, _CHUNK)],
            osems[i % _NBUF],
        )

    ins = [None] * _NCHUNK
    outs = [None] * _NCHUNK
    for i in range(min(_NBUF, _NCHUNK)):
        ins[i] = in_copy(i)
    for i in range(_NCHUNK):
        ins[i].wait()
        outs[i] = out_copy(i)
        nxt = i + _NBUF
        if nxt < _NCHUNK:
            outs[i].wait()
            ins[nxt] = in_copy(nxt)
    for i in range(max(0, _NCHUNK - _NBUF), _NCHUNK):
        outs[i].wait()


def kernel(input, embed):
    return _sc_copy(embed)


# speedup vs baseline: 1.8700x; 1.1079x over previous
import functools

import jax
import jax.numpy as jnp
from jax import lax
from jax.experimental import pallas as pl
from jax.experimental.pallas import tpu as pltpu
from jax.experimental.pallas import tpu_sc as plsc

SEQ = 8192
DIM = 1024

_NUM_CORES = 2
_NUM_SUBCORES = 16
_NW = _NUM_CORES * _NUM_SUBCORES
_ROWS_PER_W = SEQ // _NW
_CHUNK = 32
_NBUF = 3
_NCHUNK = _ROWS_PER_W // _CHUNK

_mesh = plsc.VectorSubcoreMesh(core_axis_name="c", subcore_axis_name="s")


@functools.partial(
    pl.kernel,
    mesh=_mesh,
    out_type=jax.ShapeDtypeStruct((SEQ, 1, DIM), jnp.float32),
    scratch_types=(
        [pltpu.VMEM((_CHUNK, 1, DIM), jnp.float32) for _ in range(_NBUF)]
        + [pltpu.SemaphoreType.DMA for _ in range(2 * _NBUF)]
    ),
)
def _sc_copy(embed_hbm, out_hbm, *scratch):
    bufs = scratch[:_NBUF]
    isems = scratch[_NBUF:2 * _NBUF]
    osems = scratch[2 * _NBUF:]
    wid = lax.axis_index("s") * _NUM_CORES + lax.axis_index("c")
    base = wid * _ROWS_PER_W

    def in_copy(i):
        return pltpu.async_copy(
            embed_hbm.at[pl.ds(base + i * _CHUNK, _CHUNK)],
            bufs[i % _NBUF].at[:, 0, :],
            isems[i % _NBUF],
        )

    def out_copy(i):
        return pltpu.async_copy(
            bufs[i % _NBUF],
            out_hbm.at[pl.ds(base + i * _CHUNK, _CHUNK)],
            osems[i % _NBUF],
        )

    ins = [None] * _NCHUNK
    outs = [None] * _NCHUNK
    for i in range(min(_NBUF, _NCHUNK)):
        ins[i] = in_copy(i)
    for i in range(_NCHUNK):
        ins[i].wait()
        outs[i] = out_copy(i)
        nxt = i + _NBUF
        if nxt < _NCHUNK:
            outs[i].wait()
            ins[nxt] = in_copy(nxt)
    for i in range(max(0, _NCHUNK - _NBUF), _NCHUNK):
        outs[i].wait()


def kernel(input, embed):
    return _sc_copy(embed)


# speedup vs baseline: 1.8743x; 1.0023x over previous
import functools

import jax
import jax.numpy as jnp
from jax import lax
from jax.experimental import pallas as pl
from jax.experimental.pallas import tpu as pltpu
from jax.experimental.pallas import tpu_sc as plsc

SEQ = 8192
DIM = 1024

_NUM_CORES = 2
_NUM_SUBCORES = 16
_NW = _NUM_CORES * _NUM_SUBCORES
_ROWS_PER_W = SEQ // _NW
_CHUNKS = [56, 56, 56, 56, 32]
_OFFS = [0, 56, 112, 168, 224]
_BUFROWS = 56
_NBUF = 2
_NCHUNK = len(_CHUNKS)

_mesh = plsc.VectorSubcoreMesh(core_axis_name="c", subcore_axis_name="s")


@functools.partial(
    pl.kernel,
    mesh=_mesh,
    out_type=jax.ShapeDtypeStruct((SEQ, 1, DIM), jnp.float32),
    scratch_types=(
        [pltpu.VMEM((_BUFROWS, 1, DIM), jnp.float32) for _ in range(_NBUF)]
        + [pltpu.SemaphoreType.DMA for _ in range(2 * _NBUF)]
    ),
)
def _sc_copy(embed_hbm, out_hbm, *scratch):
    bufs = scratch[:_NBUF]
    isems = scratch[_NBUF:2 * _NBUF]
    osems = scratch[2 * _NBUF:]
    wid = lax.axis_index("s") * _NUM_CORES + lax.axis_index("c")
    base = wid * _ROWS_PER_W

    def in_copy(i):
        sz = _CHUNKS[i]
        return pltpu.async_copy(
            embed_hbm.at[pl.ds(base + _OFFS[i], sz)],
            bufs[i % _NBUF].at[pl.ds(0, sz), 0, :],
            isems[i % _NBUF],
        )

    def out_copy(i):
        sz = _CHUNKS[i]
        return pltpu.async_copy(
            bufs[i % _NBUF].at[pl.ds(0, sz)],
            out_hbm.at[pl.ds(base + _OFFS[i], sz)],
            osems[i % _NBUF],
        )

    ins = [None] * _NCHUNK
    outs = [None] * _NCHUNK
    for i in range(min(_NBUF, _NCHUNK)):
        ins[i] = in_copy(i)
    for i in range(_NCHUNK):
        ins[i].wait()
        outs[i] = out_copy(i)
        nxt = i + _NBUF
        if nxt < _NCHUNK:
            outs[i].wait()
            ins[nxt] = in_copy(nxt)
    for i in range(max(0, _NCHUNK - _NBUF), _NCHUNK):
        outs[i].wait()


def kernel(input, embed):
    return _sc_copy(embed)
